# Initial kernel scaffold; baseline (speedup 1.0000x reference)
import jax
import jax.numpy as jnp
from jax.experimental import pallas as pl


def kernel(x_nchw, weight_oihw, bias):
    raise NotImplementedError("write your pallas kernel here")


# speedup vs baseline: 2.0048x; 2.0048x over previous
import jax
import jax.numpy as jnp
from jax.experimental import pallas as pl
from jax.experimental.pallas import tpu as pltpu


def _shift_rows(a, s):
    if s == 0:
        return a
    z = jnp.zeros((1, a.shape[1]), a.dtype)
    if s == 1:
        return jnp.concatenate([a[1:], z], axis=0)
    return jnp.concatenate([z, a[:-1]], axis=0)


def _shift_cols(a, s):
    if s == 0:
        return a
    z = jnp.zeros((a.shape[0], 1), a.dtype)
    if s == 1:
        return jnp.concatenate([a[:, 1:], z], axis=1)
    return jnp.concatenate([z, a[:, :-1]], axis=1)


def _conv3x3_vpu_kernel(x_ref, w_ref, b_ref, o_ref):
    _, cin, H, W = x_ref.shape
    cout = o_ref.shape[1]
    CO_CHUNK = 2

    for co0 in range(0, cout, CO_CHUNK):
        accs = [jnp.full((H, W), b_ref[co0 + j], jnp.float32)
                for j in range(CO_CHUNK)]
        for ci in range(cin):
            base = x_ref[0, ci]
            for dh in range(3):
                rs = _shift_rows(base, dh - 1)
                for dw in range(3):
                    t = _shift_cols(rs, dw - 1)
                    for j in range(CO_CHUNK):
                        accs[j] = accs[j] + t * w_ref[co0 + j, ci, dh, dw]
        for j in range(CO_CHUNK):
            o_ref[0, co0 + j] = accs[j]


@jax.jit
def _conv_impl(x_nchw, weight_oihw, bias):
    N, Cin, H, W = x_nchw.shape
    Cout = weight_oihw.shape[0]
    return pl.pallas_call(
        _conv3x3_vpu_kernel,
        out_shape=jax.ShapeDtypeStruct((N, Cout, H, W), jnp.float32),
        grid=(N,),
        in_specs=[
            pl.BlockSpec((1, Cin, H, W), lambda n: (n, 0, 0, 0)),
            pl.BlockSpec(memory_space=pltpu.SMEM),
            pl.BlockSpec(memory_space=pltpu.SMEM),
        ],
        out_specs=pl.BlockSpec((1, Cout, H, W), lambda n: (n, 0, 0, 0)),
        compiler_params=pltpu.CompilerParams(
            dimension_semantics=("parallel",),
            vmem_limit_bytes=32 * 1024 * 1024,
        ),
    )(x_nchw, weight_oihw, bias).astype(x_nchw.dtype)


def kernel(x_nchw, weight_oihw, bias):
    return _conv_impl(x_nchw, weight_oihw, bias)


# speedup vs baseline: 2.6123x; 1.3030x over previous
import jax
import jax.numpy as jnp
from jax.experimental import pallas as pl
from jax.experimental.pallas import tpu as pltpu


def _shift_rows(a, s):
    if s == 0:
        return a
    z = jnp.zeros((1, a.shape[1]), a.dtype)
    if s == 1:
        return jnp.concatenate([a[1:], z], axis=0)
    return jnp.concatenate([z, a[:-1]], axis=0)


def _shift_cols(a, s):
    if s == 0:
        return a
    z = jnp.zeros((a.shape[0], 1), a.dtype)
    if s == 1:
        return jnp.concatenate([a[:, 1:], z], axis=1)
    return jnp.concatenate([z, a[:, :-1]], axis=1)


def _conv3x3_vpu_kernel(x_ref, w_ref, b_ref, o_ref, r_ref):
    _, cin, H, W = x_ref.shape
    cout = o_ref.shape[1]

    for ci in range(cin):
        base = x_ref[0, ci]
        for dh in range(3):
            r_ref[ci * 3 + dh] = _shift_rows(base, dh - 1)

    for co in range(cout):
        acc = jnp.full((H, W), b_ref[co], jnp.float32)
        for dw in range(3):
            q = None
            for ci in range(cin):
                for dh in range(3):
                    term = r_ref[ci * 3 + dh] * w_ref[co, ci, dh, dw]
                    q = term if q is None else q + term
            acc = acc + _shift_cols(q, dw - 1)
        o_ref[0, co] = acc


@jax.jit
def _conv_impl(x_nchw, weight_oihw, bias):
    N, Cin, H, W = x_nchw.shape
    Cout = weight_oihw.shape[0]
    return pl.pallas_call(
        _conv3x3_vpu_kernel,
        out_shape=jax.ShapeDtypeStruct((N, Cout, H, W), jnp.float32),
        grid=(N,),
        in_specs=[
            pl.BlockSpec((1, Cin, H, W), lambda n: (n, 0, 0, 0)),
            pl.BlockSpec(memory_space=pltpu.SMEM),
            pl.BlockSpec(memory_space=pltpu.SMEM),
        ],
        out_specs=pl.BlockSpec((1, Cout, H, W), lambda n: (n, 0, 0, 0)),
        scratch_shapes=[pltpu.VMEM((Cin * 3, H, W), jnp.float32)],
        compiler_params=pltpu.CompilerParams(
            dimension_semantics=("parallel",),
            vmem_limit_bytes=32 * 1024 * 1024,
        ),
    )(x_nchw, weight_oihw, bias).astype(x_nchw.dtype)


def kernel(x_nchw, weight_oihw, bias):
    return _conv_impl(x_nchw, weight_oihw, bias)
